# baseline (speedup 1.0000x reference)
import functools

import jax
import jax.numpy as jnp
from jax import lax
from jax.experimental import pallas as pl
from jax.experimental.pallas import tpu as pltpu
from jax.experimental.pallas import tpu_sc as plsc

N = 10000
E = 320000
D = 128

NPAD = 10240
EPAD = 327680
K2 = EPAD // (16 * 128)
K3 = EPAD // (32 * 128)
RPS = NPAD // 16
IDXBLK = 20

_MESH = plsc.VectorSubcoreMesh(core_axis_name="c", subcore_axis_name="s")

_SCRATCH = [
    pltpu.VMEM((IDXBLK, 2, 128), jnp.int32),
    pltpu.VMEM((IDXBLK, 2, 128), jnp.int32),
    pltpu.VMEM((128, D), jnp.float32),
    pltpu.VMEM((128, D), jnp.float32),
    pltpu.VMEM_SHARED((NPAD, D), jnp.float32),
    pltpu.SemaphoreType.DMA,
    pltpu.SemaphoreType.DMA,
    pltpu.SemaphoreType.DMA,
    pltpu.SemaphoreType.DMA,
]


def _edge_loop(table_hbm, idx_view, k_chunks,
               idx0_v, idx1_v, rows0_v, rows1_v, acc,
               sem0, sem1, semi0, semi1):
    n_stages = k_chunks // IDXBLK
    banks = [(idx0_v, semi0), (idx1_v, semi1)]
    pltpu.sync_copy(idx_view.at[pl.ds(0, IDXBLK)], idx0_v)
    if n_stages > 1:
        pltpu.async_copy(idx_view.at[pl.ds(IDXBLK, IDXBLK)], idx1_v, semi1)
    for st in range(n_stages):
        idx_v, semi = banks[st % 2]
        if st > 0:
            pltpu.make_async_copy(
                idx_view.at[pl.ds(st * IDXBLK, IDXBLK)], idx_v, semi).wait()
        if st + 1 < n_stages:
            nxt_v, nsemi = banks[(st + 1) % 2]
            if st > 0:
                pltpu.async_copy(
                    idx_view.at[pl.ds((st + 1) * IDXBLK, IDXBLK)], nxt_v,
                    nsemi)
        pltpu.async_copy(table_hbm.at[idx_v.at[0, 0]], rows0_v, sem0)

        def body(i, carry):
            j = 2 * i
            pltpu.async_copy(table_hbm.at[idx_v.at[j + 1, 0]], rows1_v, sem1)
            pltpu.make_async_copy(table_hbm.at[idx_v.at[j, 0]], rows0_v,
                                  sem0).wait()
            pltpu.sync_copy(rows0_v, acc.at[idx_v.at[j, 1]], add=True)

            @pl.when(j + 2 < IDXBLK)
            def _():
                pltpu.async_copy(table_hbm.at[idx_v.at[j + 2, 0]], rows0_v,
                                 sem0)

            pltpu.make_async_copy(table_hbm.at[idx_v.at[j + 1, 0]], rows1_v,
                                  sem1).wait()
            pltpu.sync_copy(rows1_v, acc.at[idx_v.at[j + 1, 1]], add=True)
            return carry

        lax.fori_loop(0, IDXBLK // 2, body, 0)


@functools.partial(
    pl.kernel,
    out_type=[jax.ShapeDtypeStruct((NPAD, D), jnp.float32)] * 2,
    mesh=_MESH,
    scratch_types=_SCRATCH,
)
def _spmm_pass1(y1p, y2p, idx_hbm, zeros_hbm, out_x1, out_t,
                idx0_v, idx1_v, rows0_v, rows1_v, acc,
                sem0, sem1, semi0, semi1):
    c = lax.axis_index("c")
    s = lax.axis_index("s")
    sl = pl.ds(s * RPS, RPS)
    pltpu.sync_copy(zeros_hbm.at[sl], acc.at[sl])
    plsc.subcore_barrier()

    @pl.when(c == 0)
    def _():
        _edge_loop(y1p, idx_hbm.at[s], K2,
                   idx0_v, idx1_v, rows0_v, rows1_v, acc,
                   sem0, sem1, semi0, semi1)

    @pl.when(c == 1)
    def _():
        _edge_loop(y2p, idx_hbm.at[s], K2,
                   idx0_v, idx1_v, rows0_v, rows1_v, acc,
                   sem0, sem1, semi0, semi1)

    plsc.subcore_barrier()

    @pl.when(c == 0)
    def _():
        pltpu.sync_copy(acc.at[sl], out_x1.at[sl])

    @pl.when(c == 1)
    def _():
        pltpu.sync_copy(acc.at[sl], out_t.at[sl])


@functools.partial(
    pl.kernel,
    out_type=jax.ShapeDtypeStruct((2, NPAD, D), jnp.float32),
    mesh=_MESH,
    scratch_types=_SCRATCH,
)
def _spmm_pass2(t, idx_hbm, zeros_hbm, out_r,
                idx0_v, idx1_v, rows0_v, rows1_v, acc,
                sem0, sem1, semi0, semi1):
    c = lax.axis_index("c")
    s = lax.axis_index("s")
    sl = pl.ds(s * RPS, RPS)
    pltpu.sync_copy(zeros_hbm.at[sl], acc.at[sl])
    plsc.subcore_barrier()

    _edge_loop(t, idx_hbm.at[c, s], K3,
               idx0_v, idx1_v, rows0_v, rows1_v, acc,
               sem0, sem1, semi0, semi1)
    plsc.subcore_barrier()
    pltpu.sync_copy(acc.at[sl], out_r.at[c, sl])


def _mm_body(x_ref, w_ref, b_ref, o0_ref, o1_ref, o2_ref):
    y = (jnp.dot(x_ref[...], w_ref[...], preferred_element_type=jnp.float32)
         + b_ref[0, :][None, :])
    o0_ref[...] = y[:, 0:D]
    o1_ref[...] = y[:, D:2 * D]
    o2_ref[...] = y[:, 2 * D:3 * D]


def _cat_body(a_ref, b_ref, c0_ref, c1_ref, o_ref):
    o_ref[:, 0:D] = a_ref[...]
    o_ref[:, D:2 * D] = b_ref[...]
    o_ref[:, 2 * D:3 * D] = c0_ref[0] + c1_ref[0]


def kernel(x, edge_index, W0, b0, W1, b1, W2, b2):
    src = edge_index[0].astype(jnp.int32)
    dst = edge_index[1].astype(jnp.int32)
    pad = N + (jnp.arange(EPAD - E, dtype=jnp.int32) % (NPAD - N))
    srcp = jnp.concatenate([src, pad])
    dstp = jnp.concatenate([dst, pad])

    Wcat = jnp.concatenate([W0, W1, W2], axis=1)
    bcat = jnp.tile(jnp.concatenate([b0, b1, b2])[None, :], (8, 1))

    x0p, y1p, y2p = pl.pallas_call(
        _mm_body,
        grid=(10,),
        in_specs=[
            pl.BlockSpec((1024, D), lambda i: (i, 0)),
            pl.BlockSpec((D, 3 * D), lambda i: (0, 0)),
            pl.BlockSpec((8, 3 * D), lambda i: (0, 0)),
        ],
        out_specs=[pl.BlockSpec((1024, D), lambda i: (i, 0))] * 3,
        out_shape=[jax.ShapeDtypeStruct((NPAD, D), jnp.float32)] * 3,
    )(x, Wcat, bcat)

    zeros = jnp.zeros((NPAD, D), jnp.float32)

    idx = jnp.stack([srcp.reshape(16, K2, 128),
                     dstp.reshape(16, K2, 128)], axis=2)

    x1full, t = _spmm_pass1(y1p, y2p, idx, zeros)

    r = _spmm_pass2(t, idx.reshape(2, 16, K3, 2, 128), zeros)

    out = pl.pallas_call(
        _cat_body,
        grid=(10,),
        in_specs=[
            pl.BlockSpec((1000, D), lambda i: (i, 0)),
            pl.BlockSpec((1000, D), lambda i: (i, 0)),
            pl.BlockSpec((1, 1000, D), lambda i: (0, i, 0)),
            pl.BlockSpec((1, 1000, D), lambda i: (1, i, 0)),
        ],
        out_specs=pl.BlockSpec((1000, 3 * D), lambda i: (i, 0)),
        out_shape=jax.ShapeDtypeStruct((N, 3 * D), jnp.float32),
    )(x0p, x1full, r, r)
    return out


# speedup vs baseline: 1.0338x; 1.0338x over previous
import functools

import jax
import jax.numpy as jnp
from jax import lax
from jax.experimental import pallas as pl
from jax.experimental.pallas import tpu as pltpu
from jax.experimental.pallas import tpu_sc as plsc

N = 10000
E = 320000
D = 128

NPAD = 10240
EPAD = 327680
K2 = EPAD // (16 * 128)
K3 = EPAD // (32 * 128)
RPS = NPAD // 16
IDXBLK = 40

_MESH = plsc.VectorSubcoreMesh(core_axis_name="c", subcore_axis_name="s")

_SCRATCH = [
    pltpu.VMEM((IDXBLK, 128), jnp.int32),
    pltpu.VMEM((IDXBLK, 128), jnp.int32),
    pltpu.VMEM((128, D), jnp.float32),
    pltpu.VMEM((128, D), jnp.float32),
    pltpu.VMEM_SHARED((NPAD, D), jnp.float32),
    pltpu.SemaphoreType.DMA,
    pltpu.SemaphoreType.DMA,
]


def _edge_loop(table_hbm, src_view, dst_view, k_chunks,
               src_v, dst_v, rows0_v, rows1_v, acc, sem0, sem1):
    for st in range(k_chunks // IDXBLK):
        pltpu.sync_copy(src_view.at[pl.ds(st * IDXBLK, IDXBLK)], src_v)
        pltpu.sync_copy(dst_view.at[pl.ds(st * IDXBLK, IDXBLK)], dst_v)
        pltpu.async_copy(table_hbm.at[src_v.at[0]], rows0_v, sem0)

        def body(i, carry):
            j = 2 * i
            pltpu.async_copy(table_hbm.at[src_v.at[j + 1]], rows1_v, sem1)
            pltpu.make_async_copy(table_hbm.at[src_v.at[j]], rows0_v,
                                  sem0).wait()
            pltpu.sync_copy(rows0_v, acc.at[dst_v.at[j]], add=True)

            @pl.when(j + 2 < IDXBLK)
            def _():
                pltpu.async_copy(table_hbm.at[src_v.at[j + 2]], rows0_v, sem0)

            pltpu.make_async_copy(table_hbm.at[src_v.at[j + 1]], rows1_v,
                                  sem1).wait()
            pltpu.sync_copy(rows1_v, acc.at[dst_v.at[j + 1]], add=True)
            return carry

        lax.fori_loop(0, IDXBLK // 2, body, 0)


@functools.partial(
    pl.kernel,
    out_type=[jax.ShapeDtypeStruct((NPAD, D), jnp.float32)] * 2,
    mesh=_MESH,
    scratch_types=_SCRATCH,
)
def _spmm_pass1(y1p, y2p, src_hbm, dst_hbm, zeros_hbm, out_x1, out_t,
                src_v, dst_v, rows0_v, rows1_v, acc, sem0, sem1):
    c = lax.axis_index("c")
    s = lax.axis_index("s")
    sl = pl.ds(s * RPS, RPS)
    pltpu.sync_copy(zeros_hbm.at[sl], acc.at[sl])
    plsc.subcore_barrier()

    @pl.when(c == 0)
    def _():
        _edge_loop(y1p, src_hbm.at[s], dst_hbm.at[s], K2,
                   src_v, dst_v, rows0_v, rows1_v, acc, sem0, sem1)

    @pl.when(c == 1)
    def _():
        _edge_loop(y2p, src_hbm.at[s], dst_hbm.at[s], K2,
                   src_v, dst_v, rows0_v, rows1_v, acc, sem0, sem1)

    plsc.subcore_barrier()

    @pl.when(c == 0)
    def _():
        pltpu.sync_copy(acc.at[sl], out_x1.at[sl])

    @pl.when(c == 1)
    def _():
        pltpu.sync_copy(acc.at[sl], out_t.at[sl])


@functools.partial(
    pl.kernel,
    out_type=jax.ShapeDtypeStruct((2, NPAD, D), jnp.float32),
    mesh=_MESH,
    scratch_types=_SCRATCH,
)
def _spmm_pass2(t, src_hbm, dst_hbm, zeros_hbm, out_r,
                src_v, dst_v, rows0_v, rows1_v, acc, sem0, sem1):
    c = lax.axis_index("c")
    s = lax.axis_index("s")
    sl = pl.ds(s * RPS, RPS)
    pltpu.sync_copy(zeros_hbm.at[sl], acc.at[sl])
    plsc.subcore_barrier()

    _edge_loop(t, src_hbm.at[c, s], dst_hbm.at[c, s], K3,
               src_v, dst_v, rows0_v, rows1_v, acc, sem0, sem1)
    plsc.subcore_barrier()
    pltpu.sync_copy(acc.at[sl], out_r.at[c, sl])


def _mm_body(x_ref, w_ref, b_ref, o0_ref, o1_ref, o2_ref):
    y = (jnp.dot(x_ref[...], w_ref[...], preferred_element_type=jnp.float32)
         + b_ref[0, :][None, :])
    o0_ref[...] = y[:, 0:D]
    o1_ref[...] = y[:, D:2 * D]
    o2_ref[...] = y[:, 2 * D:3 * D]


def _cat_body(a_ref, b_ref, c0_ref, c1_ref, o_ref):
    o_ref[:, 0:D] = a_ref[...]
    o_ref[:, D:2 * D] = b_ref[...]
    o_ref[:, 2 * D:3 * D] = c0_ref[0] + c1_ref[0]


def kernel(x, edge_index, W0, b0, W1, b1, W2, b2):
    src = edge_index[0].astype(jnp.int32)
    dst = edge_index[1].astype(jnp.int32)
    pad = N + (jnp.arange(EPAD - E, dtype=jnp.int32) % (NPAD - N))
    srcp = jnp.concatenate([src, pad])
    dstp = jnp.concatenate([dst, pad])

    Wcat = jnp.concatenate([W0, W1, W2], axis=1)
    bcat = jnp.tile(jnp.concatenate([b0, b1, b2])[None, :], (8, 1))

    x0p, y1p, y2p = pl.pallas_call(
        _mm_body,
        grid=(10,),
        in_specs=[
            pl.BlockSpec((1024, D), lambda i: (i, 0)),
            pl.BlockSpec((D, 3 * D), lambda i: (0, 0)),
            pl.BlockSpec((8, 3 * D), lambda i: (0, 0)),
        ],
        out_specs=[pl.BlockSpec((1024, D), lambda i: (i, 0))] * 3,
        out_shape=[jax.ShapeDtypeStruct((NPAD, D), jnp.float32)] * 3,
    )(x, Wcat, bcat)

    zeros = jnp.zeros((NPAD, D), jnp.float32)

    src16 = srcp.reshape(16, K2, 128)
    dst16 = dstp.reshape(16, K2, 128)
    x1full, t = _spmm_pass1(y1p, y2p, src16, dst16, zeros)

    src32 = srcp.reshape(2, 16, K3, 128)
    dst32 = dstp.reshape(2, 16, K3, 128)
    r = _spmm_pass2(t, src32, dst32, zeros)

    out = pl.pallas_call(
        _cat_body,
        grid=(10,),
        in_specs=[
            pl.BlockSpec((1000, D), lambda i: (i, 0)),
            pl.BlockSpec((1000, D), lambda i: (i, 0)),
            pl.BlockSpec((1, 1000, D), lambda i: (0, i, 0)),
            pl.BlockSpec((1, 1000, D), lambda i: (1, i, 0)),
        ],
        out_specs=pl.BlockSpec((1000, 3 * D), lambda i: (i, 0)),
        out_shape=jax.ShapeDtypeStruct((N, 3 * D), jnp.float32),
    )(x0p, x1full, r, r)
    return out
